# baseline (speedup 1.0000x reference)
import functools

import jax
import jax.numpy as jnp
from jax import lax
from jax.experimental import pallas as pl
from jax.experimental.pallas import tpu as pltpu
from jax.experimental.pallas import tpu_sc as plsc

B = 16384
F = 26
V = 100000
D = 32
NUM_EMB = F * V
NLINE = NUM_EMB // 4

NC = 2
NS = 16
NW = NC * NS
BW = B // NW
CH = 128
NCHUNK = BW // CH

NBLK = NUM_EMB // 128
TAIL = NUM_EMB - NBLK * 128
BLK_LO = NBLK // NW
BLK_EXTRA = NBLK - BLK_LO * NW


def _transpose_block(xin, xout, ncols, iota16):
    rvec_lo = iota16 // 8
    rvec_hi = rvec_lo + 2
    lvec = lax.rem(iota16, 8)
    for q in range(ncols // 4):
        for g in range(8):
            j = 4 * q + g // 2
            rvec = rvec_lo if g % 2 == 0 else rvec_hi
            vals = plsc.load_gather(xin, [rvec, lvec, jnp.full((16,), j, jnp.int32)])
            xout[q, pl.ds((g % 2) * 16 + (g // 2) * 32, 16)] = vals


def _detile_body(tt3_hbm, t650_hbm, xin, xout):
    wid = lax.axis_index("s") * NC + lax.axis_index("c")
    start = wid * BLK_LO + jnp.minimum(wid, BLK_EXTRA)
    count = BLK_LO + jnp.where(wid < BLK_EXTRA, 1, 0)
    iota16 = lax.iota(jnp.int32, 16)

    def blk_body(i, carry):
        bk = start + i
        r0 = bk * 128
        pltpu.sync_copy(tt3_hbm.at[:, :, pl.ds(r0, 128)], xin)
        _transpose_block(xin, xout, 128, iota16)
        pltpu.sync_copy(xout, t650_hbm.at[pl.ds(bk * 32, 32)])
        return carry

    lax.fori_loop(0, count, blk_body, 0)


def _gather_body(xt_hbm, table_hbm, out_hbm, xv, acc, sem):
    wid = lax.axis_index("s") * NC + lax.axis_index("c")
    base = wid * BW

    def chunk_body(c, carry):
        cbase = base + c * CH
        pltpu.sync_copy(xt_hbm.at[:, pl.ds(cbase, CH)], xv)
        for f in range(1, F):
            off = f * V
            for i in range(CH // 16):
                sl = pl.ds(i * 16, 16)
                xv[f, sl] = xv[f, sl] + off
        pltpu.async_copy(table_hbm.at[xv.at[0]], acc, sem).wait()
        descs = [
            pltpu.async_copy(table_hbm.at[xv.at[f]], acc, sem, add=True)
            for f in range(1, F)
        ]
        for d in descs:
            d.wait()
        pltpu.sync_copy(acc, out_hbm.at[pl.ds(cbase, CH)])
        return carry

    lax.fori_loop(0, NCHUNK, chunk_body, 0)


@functools.partial(jax.jit, static_argnames=())
def kernel(x, table):
    mesh = plsc.VectorSubcoreMesh(core_axis_name="c", subcore_axis_name="s")
    tt3 = table.T.reshape(4, 8, NUM_EMB)

    detile = pl.kernel(
        _detile_body,
        out_type=jax.ShapeDtypeStruct((NLINE, 128), jnp.float32),
        mesh=mesh,
        scratch_types=[
            pltpu.VMEM((4, 8, 128), jnp.float32),
            pltpu.VMEM((32, 128), jnp.float32),
        ],
        compiler_params=pltpu.CompilerParams(
            use_tc_tiling_on_sc=True, needs_layout_passes=False
        ),
    )
    t650 = detile(tt3)
    t_lin = t650.reshape(NUM_EMB, D)
    t_lin = lax.dynamic_update_slice(
        t_lin, table[NBLK * 128 :, :], (NBLK * 128, 0)
    )

    xt = x.T
    gather = pl.kernel(
        _gather_body,
        out_type=jax.ShapeDtypeStruct((B, D), jnp.float32),
        mesh=mesh,
        scratch_types=[
            pltpu.VMEM((F, CH), jnp.int32),
            pltpu.VMEM((CH, D), jnp.float32),
            pltpu.SemaphoreType.DMA,
        ],
        compiler_params=pltpu.CompilerParams(
            use_tc_tiling_on_sc=False, needs_layout_passes=False
        ),
    )
    return gather(xt, t_lin)


# speedup vs baseline: 3.2973x; 3.2973x over previous
import functools

import jax
import jax.numpy as jnp
from jax import lax
from jax.experimental import pallas as pl
from jax.experimental.pallas import tpu as pltpu
from jax.experimental.pallas import tpu_sc as plsc

B = 16384
F = 26
V = 100000
D = 32
DP = 128
NUM_EMB = F * V

NC = 2
NS = 16
NW = NC * NS
BW = B // NW
CH = 128
NCHUNK = BW // CH


def _sc_body(xt_hbm, table_hbm, out_hbm, xv, acc, sem):
    wid = lax.axis_index("s") * NC + lax.axis_index("c")
    base = wid * BW

    def chunk_body(c, carry):
        cbase = base + c * CH
        pltpu.sync_copy(xt_hbm.at[:, pl.ds(cbase, CH)], xv)
        for f in range(1, F):
            off = f * V
            for i in range(CH // 16):
                sl = pl.ds(i * 16, 16)
                xv[f, sl] = xv[f, sl] + off
        pltpu.async_copy(table_hbm.at[xv.at[0]], acc, sem).wait()
        descs = [
            pltpu.async_copy(table_hbm.at[xv.at[f]], acc, sem, add=True)
            for f in range(1, F)
        ]
        for d in descs:
            d.wait()
        pltpu.sync_copy(acc.at[:, pl.ds(0, D)], out_hbm.at[pl.ds(cbase, CH)])
        return carry

    lax.fori_loop(0, NCHUNK, chunk_body, 0)


@functools.partial(jax.jit, static_argnames=())
def kernel(x, table):
    tpad = jnp.pad(table, ((0, 0), (0, DP - D)))
    xt = x.T
    mesh = plsc.VectorSubcoreMesh(core_axis_name="c", subcore_axis_name="s")
    run = pl.kernel(
        _sc_body,
        out_type=jax.ShapeDtypeStruct((B, D), jnp.float32),
        mesh=mesh,
        scratch_types=[
            pltpu.VMEM((F, CH), jnp.int32),
            pltpu.VMEM((CH, DP), jnp.float32),
            pltpu.SemaphoreType.DMA,
        ],
        compiler_params=pltpu.CompilerParams(
            use_tc_tiling_on_sc=False, needs_layout_passes=False
        ),
    )
    return run(xt, tpad)


# speedup vs baseline: 3.3288x; 1.0096x over previous
import functools

import jax
import jax.numpy as jnp
from jax import lax
from jax.experimental import pallas as pl
from jax.experimental.pallas import tpu as pltpu
from jax.experimental.pallas import tpu_sc as plsc

B = 16384
F = 26
V = 100000
D = 32
NUM_EMB = F * V

NC = 2
NS = 16
NW = NC * NS
BW = B // NW
CH = 128
NCHUNK = BW // CH


def _sc_body(xt_hbm, table_hbm, out_hbm, xv, acc, sem):
    wid = lax.axis_index("s") * NC + lax.axis_index("c")
    base = wid * BW

    def chunk_body(c, carry):
        cbase = base + c * CH
        pltpu.sync_copy(xt_hbm.at[:, pl.ds(cbase, CH)], xv)
        for f in range(1, F):
            off = f * V
            for i in range(CH // 16):
                sl = pl.ds(i * 16, 16)
                xv[f, sl] = xv[f, sl] + off
        pltpu.async_copy(table_hbm.at[xv.at[0]], acc, sem).wait()
        descs = [
            pltpu.async_copy(table_hbm.at[xv.at[f]], acc, sem, add=True)
            for f in range(1, F)
        ]
        for d in descs:
            d.wait()
        pltpu.sync_copy(acc, out_hbm.at[pl.ds(cbase, CH)])
        return carry

    lax.fori_loop(0, NCHUNK, chunk_body, 0)


@functools.partial(jax.jit, static_argnames=())
def kernel(x, table):
    xt = x.T
    mesh = plsc.VectorSubcoreMesh(core_axis_name="c", subcore_axis_name="s")
    run = pl.kernel(
        _sc_body,
        out_type=jax.ShapeDtypeStruct((B, D), jnp.float32),
        mesh=mesh,
        scratch_types=[
            pltpu.VMEM((F, CH), jnp.int32),
            pltpu.VMEM((CH, D), jnp.float32),
            pltpu.SemaphoreType.DMA,
        ],
        compiler_params=pltpu.CompilerParams(
            use_tc_tiling_on_sc=False, needs_layout_passes=False
        ),
    )
    return run(xt, table)
